# baseline (speedup 1.0000x reference)
import functools

import jax
import jax.numpy as jnp
from jax import lax
from jax.experimental import pallas as pl
from jax.experimental.pallas import tpu as pltpu
from jax.experimental.pallas import tpu_sc as plsc

N = 10000
D = 128
E = 320000

NC = 2
NS = 16
NW = NC * NS
NP = 10240
EB = 128
CPT = 80
CHUNKS = NW * CPT
EP = CHUNKS * EB
RPT = NP // NS
DW = 128

_MESH = plsc.VectorSubcoreMesh(core_axis_name="c", subcore_axis_name="s")


@functools.partial(
    pl.kernel,
    out_type=jax.ShapeDtypeStruct((NC * NP, DW), jnp.float32),
    mesh=_MESH,
    scratch_types=[
        pltpu.VMEM((CPT, EB), jnp.int32),
        pltpu.VMEM((EB, DW), jnp.float32),
        pltpu.VMEM_SHARED((NP, DW), jnp.float32),
        pltpu.SemaphoreType.DMA,
    ],
)
def _deg_kernel(dst_hbm, ones_hbm, zeros_hbm, deg_out, dst_v, ones_v, acc,
                dsem):
    cid = lax.axis_index("c")
    sid = lax.axis_index("s")
    wid = sid * NC + cid
    pltpu.sync_copy(zeros_hbm.at[pl.ds(sid * RPT, RPT)],
                    acc.at[pl.ds(sid * RPT, RPT)])
    pltpu.sync_copy(ones_hbm, ones_v)
    pltpu.sync_copy(dst_hbm.at[pl.ds(wid * CPT, CPT)], dst_v)
    plsc.subcore_barrier()

    @pl.loop(0, CPT)
    def _(j):
        pltpu.sync_copy(ones_v, acc.at[dst_v.at[j]], add=True)

    plsc.subcore_barrier()
    pltpu.sync_copy(acc.at[pl.ds(sid * RPT, RPT)],
                    deg_out.at[pl.ds(cid * NP + sid * RPT, RPT)])


CR = 2
OPT = CPT // CR
IB = 20


@functools.partial(
    pl.kernel,
    out_type=jax.ShapeDtypeStruct((NC * NP, D), jnp.float32),
    mesh=_MESH,
    scratch_types=[
        pltpu.VMEM((IB * CR * EB,), jnp.int32),
        pltpu.VMEM((IB * CR, EB), jnp.int32),
        pltpu.VMEM((CR * EB, D), jnp.float32),
        pltpu.VMEM_SHARED((NP, D), jnp.float32),
        pltpu.SemaphoreType.DMA,
    ],
)
def _agg_kernel(src_hbm, dst_hbm, y_hbm, zeros_hbm, agg_out,
                si_v, di_v, rows_v, acc, gsem):
    cid = lax.axis_index("c")
    sid = lax.axis_index("s")
    wid = sid * NC + cid
    pltpu.sync_copy(zeros_hbm.at[pl.ds(sid * RPT, RPT)],
                    acc.at[pl.ds(sid * RPT, RPT)])
    plsc.subcore_barrier()

    @pl.loop(0, OPT // IB)
    def _(blk):
        base = wid * OPT + blk * IB
        pltpu.sync_copy(src_hbm.at[pl.ds(base * CR * EB, IB * CR * EB)], si_v)
        pltpu.sync_copy(dst_hbm.at[pl.ds(base * CR, IB * CR)], di_v)

        @pl.loop(0, IB)
        def _(jj):
            pltpu.async_copy(y_hbm.at[si_v.at[pl.ds(jj * CR * EB, CR * EB)]],
                             rows_v, gsem).wait()
            for r in range(CR):
                pltpu.sync_copy(rows_v.at[pl.ds(r * EB, EB)],
                                acc.at[di_v.at[jj * CR + r]], add=True)

    plsc.subcore_barrier()
    pltpu.sync_copy(acc.at[pl.ds(sid * RPT, RPT)],
                    agg_out.at[pl.ds(cid * NP + sid * RPT, RPT)])


def _mm_body(x_ref, w_ref, dg0_ref, dg1_ref, y_ref):
    deg = dg0_ref[:, 0:1] + dg1_ref[:, 0:1] + 1.0
    dinv = lax.rsqrt(deg)
    xw = jnp.dot(x_ref[...], w_ref[...], preferred_element_type=jnp.float32)
    y_ref[...] = xw * dinv


def _final_body(a0_ref, a1_ref, y_ref, dg0_ref, dg1_ref, b_ref,
                wt_ref, t_ref, o_ref):
    deg = dg0_ref[:, 0:1] + dg1_ref[:, 0:1] + 1.0
    dinv = lax.rsqrt(deg)
    out = dinv * (a0_ref[...] + a1_ref[...] + y_ref[...]) + b_ref[...]
    t = t_ref[0]
    s = out / t
    m = jnp.max(s, axis=1, keepdims=True)
    e = jnp.exp(s - m)
    z = jnp.sum(e, axis=1, keepdims=True)
    p = e / z
    logp = jnp.log(p + 1e-12)
    h = -jnp.sum(p * logp, axis=1, keepdims=True)
    eg = -(p * (logp + h)) / t
    o_ref[...] = out + wt_ref[0] * eg


_RB = 1000


def kernel(x, edge_index, W, b, weight, temperature):
    ei = edge_index.astype(jnp.int32)
    pad = jnp.full((2, EP - E), N, jnp.int32)
    eip = jnp.concatenate([ei, pad], axis=1)
    src1d = eip[0]
    dst3d = eip[1].reshape(CHUNKS, EB)
    dst2d = eip[1].reshape(CHUNKS, EB)
    ones8 = jnp.ones((EB, DW), jnp.float32)
    zeros8 = jnp.zeros((NP, DW), jnp.float32)
    zerosD = jnp.zeros((NP, D), jnp.float32)

    deg_part = _deg_kernel(dst2d, ones8, zeros8)
    dg0, dg1 = deg_part[:N], deg_part[NP:NP + N]

    y = pl.pallas_call(
        _mm_body,
        grid=(N // _RB,),
        in_specs=[
            pl.BlockSpec((_RB, D), lambda i: (i, 0)),
            pl.BlockSpec((D, D), lambda i: (0, 0)),
            pl.BlockSpec((_RB, DW), lambda i: (i, 0)),
            pl.BlockSpec((_RB, DW), lambda i: (i, 0)),
        ],
        out_specs=pl.BlockSpec((_RB, D), lambda i: (i, 0)),
        out_shape=jax.ShapeDtypeStruct((N, D), jnp.float32),
    )(x, W, dg0, dg1)

    y_pad = jnp.concatenate([y, jnp.zeros((NP - N, D), jnp.float32)])
    agg_part = _agg_kernel(src1d, dst3d, y_pad, zerosD)
    a0, a1 = agg_part[:N], agg_part[NP:NP + N]

    out = pl.pallas_call(
        _final_body,
        grid=(N // _RB,),
        in_specs=[
            pl.BlockSpec((_RB, D), lambda i: (i, 0)),
            pl.BlockSpec((_RB, D), lambda i: (i, 0)),
            pl.BlockSpec((_RB, D), lambda i: (i, 0)),
            pl.BlockSpec((_RB, DW), lambda i: (i, 0)),
            pl.BlockSpec((_RB, DW), lambda i: (i, 0)),
            pl.BlockSpec((1, D), lambda i: (0, 0)),
            pl.BlockSpec(memory_space=pltpu.SMEM),
            pl.BlockSpec(memory_space=pltpu.SMEM),
        ],
        out_specs=pl.BlockSpec((_RB, D), lambda i: (i, 0)),
        out_shape=jax.ShapeDtypeStruct((N, D), jnp.float32),
    )(a0, a1, y, dg0, dg1, b.reshape(1, D),
      weight.reshape(1), temperature.reshape(1))
    return out


# speedup vs baseline: 1.1836x; 1.1836x over previous
import functools

import jax
import jax.numpy as jnp
from jax import lax
from jax.experimental import pallas as pl
from jax.experimental.pallas import tpu as pltpu
from jax.experimental.pallas import tpu_sc as plsc

N = 10000
D = 128
E = 320000

NC = 2
NS = 16
NW = NC * NS
NP = 10240
EB = 128
CPT = 80
CHUNKS = NW * CPT
EP = CHUNKS * EB
RPT = NP // NS
DW = 128

_MESH = plsc.VectorSubcoreMesh(core_axis_name="c", subcore_axis_name="s")


@functools.partial(
    pl.kernel,
    out_type=jax.ShapeDtypeStruct((NC * NP, DW), jnp.float32),
    mesh=_MESH,
    scratch_types=[
        pltpu.VMEM((CPT, EB), jnp.int32),
        pltpu.VMEM((EB, DW), jnp.float32),
        pltpu.VMEM_SHARED((NP, DW), jnp.float32),
        pltpu.SemaphoreType.DMA,
    ],
)
def _deg_kernel(dst_hbm, ones_hbm, zeros_hbm, deg_out, dst_v, ones_v, acc,
                dsem):
    cid = lax.axis_index("c")
    sid = lax.axis_index("s")
    wid = sid * NC + cid
    pltpu.sync_copy(zeros_hbm.at[pl.ds(sid * RPT, RPT)],
                    acc.at[pl.ds(sid * RPT, RPT)])
    pltpu.sync_copy(ones_hbm, ones_v)
    pltpu.sync_copy(dst_hbm.at[pl.ds(wid * CPT, CPT)], dst_v)
    plsc.subcore_barrier()

    @pl.loop(0, CPT)
    def _(j):
        pltpu.sync_copy(ones_v, acc.at[dst_v.at[j]], add=True)

    plsc.subcore_barrier()
    pltpu.sync_copy(acc.at[pl.ds(sid * RPT, RPT)],
                    deg_out.at[pl.ds(cid * NP + sid * RPT, RPT)])


NB = 2
IB = 8
CPT0 = 136
CPT1 = 24


@functools.partial(
    pl.kernel,
    out_type=jax.ShapeDtypeStruct((NC * NP, D), jnp.float32),
    mesh=_MESH,
    scratch_types=[
        pltpu.VMEM((IB, EB), jnp.int32),
        pltpu.VMEM((IB, EB), jnp.int32),
        pltpu.VMEM((NB, EB, D), jnp.float32),
        pltpu.VMEM_SHARED((NP, D), jnp.float32),
        pltpu.SemaphoreType.DMA,
        pltpu.SemaphoreType.DMA,
        pltpu.SemaphoreType.DMA,
        pltpu.SemaphoreType.DMA,
    ],
)
def _agg_kernel(src_hbm, dst_hbm, y_hbm, zeros_hbm, agg_out,
                si_v, di_v, rows_v, acc, gs0, gs1, ss0, ss1):
    gsem = [gs0, gs1]
    ssem = [ss0, ss1]
    cid = lax.axis_index("c")
    sid = lax.axis_index("s")
    cptw = jnp.where(cid == 0, CPT0, CPT1)
    cbase = jnp.where(cid == 0, sid * CPT0, NS * CPT0 + sid * CPT1)
    pltpu.sync_copy(zeros_hbm.at[pl.ds(sid * RPT, RPT)],
                    acc.at[pl.ds(sid * RPT, RPT)])
    plsc.subcore_barrier()

    @pl.loop(0, cptw // IB)
    def _(blk):
        base = cbase + blk * IB
        pltpu.sync_copy(src_hbm.at[pl.ds(base, IB)], si_v)
        pltpu.sync_copy(dst_hbm.at[pl.ds(base, IB)], di_v)
        pltpu.async_copy(y_hbm.at[si_v.at[0]], rows_v.at[0], gsem[0])

        @pl.loop(0, IB, step=NB)
        def _(g):
            for b in range(NB):
                jj = g + b
                nxt = jj + 1
                bn = 1 - b

                @pl.when(jnp.logical_and(nxt >= 2, nxt < IB))
                def _():
                    pltpu.make_async_copy(rows_v.at[bn],
                                          acc.at[di_v.at[nxt - 2]],
                                          ssem[bn]).wait()
                    pltpu.async_copy(y_hbm.at[si_v.at[nxt]], rows_v.at[bn],
                                     gsem[bn])

                @pl.when(nxt == 1)
                def _():
                    pltpu.async_copy(y_hbm.at[si_v.at[nxt]], rows_v.at[bn],
                                     gsem[bn])

                pltpu.make_async_copy(y_hbm.at[si_v.at[jj]], rows_v.at[b],
                                      gsem[b]).wait()
                pltpu.async_copy(rows_v.at[b], acc.at[di_v.at[jj]], ssem[b],
                                 add=True)

        for b in range(NB):
            pltpu.make_async_copy(rows_v.at[b],
                                  acc.at[di_v.at[IB - NB + b]],
                                  ssem[b]).wait()

    plsc.subcore_barrier()
    pltpu.sync_copy(acc.at[pl.ds(sid * RPT, RPT)],
                    agg_out.at[pl.ds(cid * NP + sid * RPT, RPT)])


def _mm_body(x_ref, w_ref, dg0_ref, dg1_ref, y_ref):
    deg = dg0_ref[:, 0:1] + dg1_ref[:, 0:1] + 1.0
    dinv = lax.rsqrt(deg)
    xw = jnp.dot(x_ref[...], w_ref[...], preferred_element_type=jnp.float32)
    y_ref[...] = xw * dinv


def _final_body(a0_ref, a1_ref, y_ref, dg0_ref, dg1_ref, b_ref,
                wt_ref, t_ref, o_ref):
    deg = dg0_ref[:, 0:1] + dg1_ref[:, 0:1] + 1.0
    dinv = lax.rsqrt(deg)
    out = dinv * (a0_ref[...] + a1_ref[...] + y_ref[...]) + b_ref[...]
    t = t_ref[0]
    s = out / t
    m = jnp.max(s, axis=1, keepdims=True)
    e = jnp.exp(s - m)
    z = jnp.sum(e, axis=1, keepdims=True)
    p = e / z
    logp = jnp.log(p + 1e-12)
    h = -jnp.sum(p * logp, axis=1, keepdims=True)
    eg = -(p * (logp + h)) / t
    o_ref[...] = out + wt_ref[0] * eg


_RB = 1000


def kernel(x, edge_index, W, b, weight, temperature):
    ei = edge_index.astype(jnp.int32)
    pad = jnp.full((2, EP - E), N, jnp.int32)
    eip = jnp.concatenate([ei, pad], axis=1)
    src2d = eip[0].reshape(CHUNKS, EB)
    dst2d = eip[1].reshape(CHUNKS, EB)
    ones8 = jnp.ones((EB, DW), jnp.float32)
    zeros8 = jnp.zeros((NP, DW), jnp.float32)
    zerosD = jnp.zeros((NP, D), jnp.float32)

    deg_part = _deg_kernel(dst2d, ones8, zeros8)
    dg0, dg1 = deg_part[:N], deg_part[NP:NP + N]

    y = pl.pallas_call(
        _mm_body,
        grid=(N // _RB,),
        in_specs=[
            pl.BlockSpec((_RB, D), lambda i: (i, 0)),
            pl.BlockSpec((D, D), lambda i: (0, 0)),
            pl.BlockSpec((_RB, DW), lambda i: (i, 0)),
            pl.BlockSpec((_RB, DW), lambda i: (i, 0)),
        ],
        out_specs=pl.BlockSpec((_RB, D), lambda i: (i, 0)),
        out_shape=jax.ShapeDtypeStruct((N, D), jnp.float32),
    )(x, W, dg0, dg1)

    y_pad = jnp.concatenate([y, jnp.zeros((NP - N, D), jnp.float32)])
    agg_part = _agg_kernel(src2d, dst2d, y_pad, zerosD)
    a0, a1 = agg_part[:N], agg_part[NP:NP + N]

    out = pl.pallas_call(
        _final_body,
        grid=(N // _RB,),
        in_specs=[
            pl.BlockSpec((_RB, D), lambda i: (i, 0)),
            pl.BlockSpec((_RB, D), lambda i: (i, 0)),
            pl.BlockSpec((_RB, D), lambda i: (i, 0)),
            pl.BlockSpec((_RB, DW), lambda i: (i, 0)),
            pl.BlockSpec((_RB, DW), lambda i: (i, 0)),
            pl.BlockSpec((1, D), lambda i: (0, 0)),
            pl.BlockSpec(memory_space=pltpu.SMEM),
            pl.BlockSpec(memory_space=pltpu.SMEM),
        ],
        out_specs=pl.BlockSpec((_RB, D), lambda i: (i, 0)),
        out_shape=jax.ShapeDtypeStruct((N, D), jnp.float32),
    )(a0, a1, y, dg0, dg1, b.reshape(1, D),
      weight.reshape(1), temperature.reshape(1))
    return out
